# Initial kernel scaffold; baseline (speedup 1.0000x reference)
import jax
import jax.numpy as jnp
from jax.experimental import pallas as pl


def kernel(h_id, r_id, src, dst, rel_id, ent_emb, gate, rel_embs, S_w, S_b, L_w, L_b, W, W_r, a, neigh_w, conv_w, conv_b, fc_w, fc_b, score_b):
    raise NotImplementedError("write your pallas kernel here")


# speedup vs baseline: 23.9814x; 23.9814x over previous
import dataclasses
import functools

import jax
import jax.numpy as jnp
from jax.experimental import pallas as pl
from jax.experimental.pallas import tpu as pltpu
from jax.experimental.pallas import tpu_sc as plsc

N_ENT = 10000
N_REL = 200
NRE = 2 * N_REL
H = 128
DEG = 32
TOPK = 8
MAX_N = 3
NCC = 2 * MAX_N
E = N_ENT * DEG
BS = 1024
OUT_CH = 32
KER = 7
KH = 8
KW = 16
CH = 2 * KH - KER + 1
CW = KW - KER + 1
FC_IN = OUT_CH * CH * CW

K1_BLK = 1000
K2_BLK = 400
K3_BLK = 400
K4_BLK = 256
GATHER_WIN = 128


def _leaky(x):
    return jnp.where(x > 0, x, 0.2 * x)


def _sc_compiler_params():
    cp = pltpu.CompilerParams()
    if "needs_layout_passes" in pltpu.CompilerParams.__dataclass_fields__:
        cp = dataclasses.replace(cp, needs_layout_passes=False)
    return cp


def _sc_gather1(table, idx, window=GATHER_WIN):
    n = idx.shape[0]
    vd = table.shape[1]
    info = plsc.get_sparse_core_info()
    nw = info.num_cores * info.num_subcores
    quantum = nw * window
    n_pad = ((n + quantum - 1) // quantum) * quantum
    if n_pad != n:
        idx = jnp.pad(idx, (0, n_pad - n))
    per_w = n_pad // nw
    nsteps = per_w // window
    mesh = plsc.VectorSubcoreMesh(core_axis_name="c", subcore_axis_name="s")

    @functools.partial(
        pl.kernel,
        out_type=jax.ShapeDtypeStruct((n_pad, vd), table.dtype),
        mesh=mesh,
        scratch_types=[
            pltpu.VMEM((window,), jnp.int32),
            pltpu.VMEM((window, vd), table.dtype),
        ])
    def gk(t_hbm, i_hbm, o_hbm, i_v, r_v):
        wid = jax.lax.axis_index("s") * info.num_cores + jax.lax.axis_index("c")
        base = wid * per_w

        @pl.loop(0, nsteps)
        def _(step):
            cb = base + step * window
            pltpu.sync_copy(i_hbm.at[pl.ds(cb, window)], i_v)
            pltpu.sync_copy(t_hbm.at[i_v], r_v)
            pltpu.sync_copy(r_v, o_hbm.at[pl.ds(cb, window)])

    return gk(table, idx)


def _sc_scores(ns6, rs6, src, rel):
    n = src.shape[0]
    nwu = 25
    per_w = n // nwu
    chunk = 3200
    nsteps = per_w // chunk
    mesh = plsc.VectorSubcoreMesh(core_axis_name="c", subcore_axis_name="s")
    info = plsc.get_sparse_core_info()

    @functools.partial(
        pl.kernel,
        out_type=jax.ShapeDtypeStruct((nwu * NCC, per_w), jnp.float32),
        mesh=mesh,
        compiler_params=_sc_compiler_params(),
        scratch_types=(
            [pltpu.VMEM((N_ENT,), jnp.float32)] * NCC
            + [pltpu.VMEM((NRE,), jnp.float32)] * NCC
            + [pltpu.VMEM((chunk,), jnp.int32)] * 2
            + [pltpu.VMEM((chunk,), jnp.float32)] * NCC
        ))
    def sk(*args):
        (ns_hbm, rs_hbm, src_hbm, rel_hbm, q_hbm) = args[:5]
        sc = args[5:]
        ns_v = sc[:NCC]
        rs_v = sc[NCC:2 * NCC]
        src_v, rel_v = sc[2 * NCC:2 * NCC + 2]
        q_v = sc[2 * NCC + 2:]
        wid = jax.lax.axis_index("s") * info.num_cores + jax.lax.axis_index("c")

        @pl.when(wid < nwu)
        def _():
            for cc in range(NCC):
                pltpu.sync_copy(ns_hbm.at[cc], ns_v[cc])
                pltpu.sync_copy(rs_hbm.at[cc], rs_v[cc])

            @pl.loop(0, nsteps)
            def _(step):
                cbase = wid * per_w + step * chunk
                pltpu.sync_copy(src_hbm.at[pl.ds(cbase, chunk)], src_v)
                pltpu.sync_copy(rel_hbm.at[pl.ds(cbase, chunk)], rel_v)

                @pl.loop(0, chunk, step=16)
                def _(o):
                    sidx = src_v[pl.ds(o, 16)]
                    ridx = rel_v[pl.ds(o, 16)]
                    for cc in range(NCC):
                        qv = (plsc.load_gather(ns_v[cc], [sidx])
                              + plsc.load_gather(rs_v[cc], [ridx]))
                        q_v[cc][pl.ds(o, 16)] = qv

                for cc in range(NCC):
                    pltpu.sync_copy(
                        q_v[cc],
                        q_hbm.at[wid * NCC + cc, pl.ds(step * chunk, chunk)])

    q = sk(ns6, rs6, src, rel)
    return (q.reshape(nwu, NCC, per_w).transpose(1, 0, 2).reshape(NCC, n))


def _sc_gather2(tab1, idx1, tab2, idx2, window=GATHER_WIN):
    n = idx1.shape[0]
    vd1, vd2 = tab1.shape[1], tab2.shape[1]
    info = plsc.get_sparse_core_info()
    nw = info.num_cores * info.num_subcores
    per_w = n // nw
    assert n % (nw * window) == 0, (n, window)
    nsteps = per_w // window
    mesh = plsc.VectorSubcoreMesh(core_axis_name="c", subcore_axis_name="s")

    @functools.partial(
        pl.kernel,
        out_type=(jax.ShapeDtypeStruct((n, vd1), tab1.dtype),
                  jax.ShapeDtypeStruct((n, vd2), tab2.dtype)),
        mesh=mesh,
        scratch_types=[
            pltpu.VMEM((window,), jnp.int32),
            pltpu.VMEM((window, vd1), tab1.dtype),
            pltpu.VMEM((window,), jnp.int32),
            pltpu.VMEM((window, vd2), tab2.dtype),
        ])
    def gk(t1_hbm, i1_hbm, t2_hbm, i2_hbm, o1_hbm, o2_hbm,
           i1_v, r1_v, i2_v, r2_v):
        wid = jax.lax.axis_index("s") * info.num_cores + jax.lax.axis_index("c")
        base = wid * per_w

        @pl.loop(0, nsteps)
        def _(step):
            cb = base + step * window
            pltpu.sync_copy(i1_hbm.at[pl.ds(cb, window)], i1_v)
            pltpu.sync_copy(t1_hbm.at[i1_v], r1_v)
            pltpu.sync_copy(r1_v, o1_hbm.at[pl.ds(cb, window)])
            pltpu.sync_copy(i2_hbm.at[pl.ds(cb, window)], i2_v)
            pltpu.sync_copy(t2_hbm.at[i2_v], r2_v)
            pltpu.sync_copy(r2_v, o2_hbm.at[pl.ds(cb, window)])

    return gk(tab1, idx1, tab2, idx2)


def _k1_body(ent_ref, sw_ref, sb_ref, lw_ref, lb_ref, w_ref, wr_ref,
             a12_ref, a3_ref, re_ref, eg_ref,
             h_ref, nsnd_ref, rs_ref, rtab_ref, prel_ref):
    ent = ent_ref[...]
    common = jnp.dot(ent, sw_ref[...], preferred_element_type=jnp.float32) + sb_ref[...]
    private = jnp.dot(ent, lw_ref[...], preferred_element_type=jnp.float32) + lb_ref[...]
    ns_cols = []
    nd_cols = []
    for i in range(MAX_N):
        a12 = a12_ref[i]
        for ch, x in ((0, common), (1, private)):
            cc = 2 * i + ch
            h = jnp.dot(x, w_ref[i], preferred_element_type=jnp.float32)
            h_ref[cc, :, :] = h
            nsnd = jnp.dot(h, a12, preferred_element_type=jnp.float32)
            ns_cols.append(nsnd[:, 0:1])
            nd_cols.append(nsnd[:, 1:2])
    pad = jnp.zeros((ent.shape[0], 16 - 2 * NCC), jnp.float32)
    nsnd_ref[...] = jnp.concatenate(ns_cols + nd_cols + [pad], axis=1)

    @pl.when(pl.program_id(0) == 0)
    def _():
        rs_cols = []
        for i in range(MAX_N):
            rt = jnp.dot(re_ref[i], wr_ref[i], preferred_element_type=jnp.float32)
            rtab_ref[i, :, :] = rt
            rs = jnp.dot(rt, a3_ref[i], preferred_element_type=jnp.float32)
            rs_cols.append(rs)
            rs_cols.append(rs)
        rpad = jnp.zeros((NRE, 16 - NCC), jnp.float32)
        rs_ref[...] = jnp.concatenate(rs_cols + [rpad], axis=1)
        prel_ref[...] = (eg_ref[0, 0] * re_ref[0] + eg_ref[0, 1] * re_ref[1]
                         + eg_ref[0, 2] * re_ref[2])


def _k2_body(q_ref, nsnd_ref, src_ref, rel_ref,
             ss_ref, sr_ref, w_ref):
    blk = K2_BLK
    nsnd = nsnd_ref[...]
    src2d = src_ref[...]
    rel2d = rel_ref[...]
    ss_cols = []
    sr_cols = []
    w_cols = []
    lane = jax.lax.broadcasted_iota(jnp.int32, (blk, DEG), 1)
    for cc in range(NCC):
        s = _leaky(q_ref[cc] + nsnd[:, NCC + cc][:, None])
        rank = jnp.zeros((blk, DEG), jnp.int32)
        for k in range(DEG):
            sk = s[:, k:k + 1]
            beats = (sk > s) | ((sk == s) & (k < lane))
            rank = rank + beats.astype(jnp.int32)
        sel = rank < TOPK
        sm = jnp.where(sel, s, -1e30)
        mx = sm.max(axis=1, keepdims=True)
        p = jnp.where(sel, jnp.exp(sm - mx), 0.0)
        attn = p / p.sum(axis=1, keepdims=True)
        w8 = []
        ss8 = []
        sr8 = []
        for k in range(TOPK):
            oh = rank == k
            w8.append(jnp.where(oh, attn, 0.0).sum(axis=1, keepdims=True))
            ss8.append(jnp.where(oh, src2d, 0).sum(axis=1, keepdims=True))
            sr8.append(jnp.where(oh, rel2d, 0).sum(axis=1, keepdims=True))
        ss_cols.append(jnp.concatenate(ss8, axis=1) + cc * N_ENT)
        sr_cols.append(jnp.concatenate(sr8, axis=1))
        w_cols.append(jnp.concatenate(w8, axis=1))
    ss_ref[...] = jnp.concatenate(ss_cols, axis=1)
    sr_ref[...] = jnp.concatenate(sr_cols, axis=1)
    w_ref[...] = jnp.concatenate(w_cols, axis=1)


def _k3_body(gh_ref, w_ref, sr_ref, ent_ref, rtab_ref, nw_ref, eg_ref,
             fused_ref, c1_ref, csq_ref, ccr_ref):
    blk = K3_BLK
    G = gh_ref[...].reshape(blk, NCC * TOPK, H)
    w = w_ref[...]
    sr = sr_ref[...]

    @pl.when(pl.program_id(0) == 0)
    def _():
        c1_ref[...] = jnp.zeros_like(c1_ref)
        csq_ref[...] = jnp.zeros_like(csq_ref)
        ccr_ref[...] = jnp.zeros_like(ccr_ref)

    fused = ent_ref[...]
    iota_r = jax.lax.broadcasted_iota(jnp.int32, (blk, NRE), 1)
    for i in range(MAX_N):
        couts = []
        for ch in range(2):
            cc = 2 * i + ch
            wcc = w[:, cc * TOPK:(cc + 1) * TOPK]
            neigh = (G[:, cc * TOPK:(cc + 1) * TOPK, :] * wcc[:, :, None]).sum(axis=1)
            A = jnp.zeros((blk, NRE), jnp.float32)
            for k in range(TOPK):
                col = cc * TOPK + k
                A = A + jnp.where(sr[:, col][:, None] == iota_r,
                                  w[:, col][:, None], 0.0)
            neigh = neigh + jnp.dot(A, rtab_ref[i], preferred_element_type=jnp.float32)
            cout = jnp.tanh(jnp.dot(neigh, nw_ref[i], preferred_element_type=jnp.float32))
            couts.append(cout)
            fused = fused + eg_ref[0, i] * cout
            c1_ref[cc:cc + 1, :] += cout.sum(axis=0, keepdims=True)
            csq_ref[cc:cc + 1, :] += (cout * cout).sum(axis=0, keepdims=True)
        ccr_ref[i:i + 1, :] += (couts[0] * couts[1]).sum(axis=0, keepdims=True)
    fused_ref[...] = fused


def _k4_body(head_ref, rel_ref, m_ref, bconv_ref, fcw_ref, fcb_ref,
             fusedt_ref, sb_ref, out_ref):
    x = jnp.concatenate([head_ref[...], rel_ref[...]], axis=1)
    y1 = jnp.maximum(
        jnp.dot(x, m_ref[...], preferred_element_type=jnp.float32) + bconv_ref[...], 0.0)
    y2 = jnp.maximum(
        jnp.dot(y1, fcw_ref[...], preferred_element_type=jnp.float32) + fcb_ref[...], 0.0)
    out_ref[...] = (jnp.dot(y2, fusedt_ref[...], preferred_element_type=jnp.float32)
                    + sb_ref[...])


def _build_conv_mat(conv_w):
    py = jnp.arange(2 * KH)[:, None]
    oy = jnp.arange(CH)[None, :]
    dy = py - oy
    px = jnp.arange(KW)[:, None]
    ox = jnp.arange(CW)[None, :]
    dx = px - ox
    ok = (dy >= 0) & (dy < KER)
    okx = (dx >= 0) & (dx < KER)
    wy = jnp.clip(dy, 0, KER - 1)
    wx = jnp.clip(dx, 0, KER - 1)
    M = conv_w[:, 0][:, wy][:, :, :, wx]
    M = M * (ok[None, :, :, None, None] & okx[None, None, None, :, :])
    M = M.transpose(1, 3, 0, 2, 4)
    return M.reshape(2 * KH * KW, OUT_CH * CH * CW)


def kernel(h_id, r_id, src, dst, rel_id, ent_emb, gate, rel_embs,
           S_w, S_b, L_w, L_b, W, W_r, a, neigh_w,
           conv_w, conv_b, fc_w, fc_b, score_b):
    f32 = jnp.float32
    ent_emb = ent_emb.astype(f32)
    src = src.astype(jnp.int32)
    rel_id = rel_id.astype(jnp.int32)
    h_id = h_id.astype(jnp.int32)
    r_id = r_id.astype(jnp.int32)

    gw = jax.nn.softmax(gate.astype(f32))
    mask = gw > 0.1
    eff = jnp.where(mask.any(), mask, jnp.arange(MAX_N) == jnp.argmax(gw))
    m = eff.astype(f32)
    eg = gw * m
    eg = eg / eg.sum()
    eg_v = jnp.zeros((1, 128), f32).at[0, :MAX_N].set(eg)

    a_m = a[:, :, 0].astype(f32)
    a12 = jnp.stack([a_m[:, :H], a_m[:, H:2 * H]], axis=2)
    a3 = a_m[:, 2 * H:][:, :, None]
    sb2 = S_b.reshape(1, H).astype(f32)
    lb2 = L_b.reshape(1, H).astype(f32)

    grid1 = N_ENT // K1_BLK
    h_all, nsnd, rs_tab, r_tabs, pred_rel = pl.pallas_call(
        _k1_body,
        grid=(grid1,),
        in_specs=[
            pl.BlockSpec((K1_BLK, H), lambda b: (b, 0)),
            pl.BlockSpec((H, H), lambda b: (0, 0)),
            pl.BlockSpec((1, H), lambda b: (0, 0)),
            pl.BlockSpec((H, H), lambda b: (0, 0)),
            pl.BlockSpec((1, H), lambda b: (0, 0)),
            pl.BlockSpec((MAX_N, H, H), lambda b: (0, 0, 0)),
            pl.BlockSpec((MAX_N, H, H), lambda b: (0, 0, 0)),
            pl.BlockSpec((MAX_N, H, 2), lambda b: (0, 0, 0)),
            pl.BlockSpec((MAX_N, H, 1), lambda b: (0, 0, 0)),
            pl.BlockSpec((MAX_N, NRE, H), lambda b: (0, 0, 0)),
            pl.BlockSpec((1, 128), lambda b: (0, 0)),
        ],
        out_specs=[
            pl.BlockSpec((NCC, K1_BLK, H), lambda b: (0, b, 0)),
            pl.BlockSpec((K1_BLK, 16), lambda b: (b, 0)),
            pl.BlockSpec((NRE, 16), lambda b: (0, 0)),
            pl.BlockSpec((MAX_N, NRE, H), lambda b: (0, 0, 0)),
            pl.BlockSpec((NRE, H), lambda b: (0, 0)),
        ],
        out_shape=[
            jax.ShapeDtypeStruct((NCC, N_ENT, H), f32),
            jax.ShapeDtypeStruct((N_ENT, 16), f32),
            jax.ShapeDtypeStruct((NRE, 16), f32),
            jax.ShapeDtypeStruct((MAX_N, NRE, H), f32),
            jax.ShapeDtypeStruct((NRE, H), f32),
        ],
    )(ent_emb, S_w.astype(f32), sb2, L_w.astype(f32), lb2,
      W.astype(f32), W_r.astype(f32), a12, a3, rel_embs.astype(f32), eg_v)

    ns6 = nsnd[:, :NCC].T
    rs6 = rs_tab[:, :NCC].T
    q = _sc_scores(ns6, rs6, src, rel_id).reshape(NCC, N_ENT, DEG)

    grid2 = N_ENT // K2_BLK
    src2d = src.reshape(N_ENT, DEG)
    rel2d = rel_id.reshape(N_ENT, DEG)
    ss, sr, wsel = pl.pallas_call(
        _k2_body,
        grid=(grid2,),
        in_specs=[
            pl.BlockSpec((NCC, K2_BLK, DEG), lambda b: (0, b, 0)),
            pl.BlockSpec((K2_BLK, 16), lambda b: (b, 0)),
            pl.BlockSpec((K2_BLK, DEG), lambda b: (b, 0)),
            pl.BlockSpec((K2_BLK, DEG), lambda b: (b, 0)),
        ],
        out_specs=[
            pl.BlockSpec((K2_BLK, NCC * TOPK), lambda b: (b, 0)),
            pl.BlockSpec((K2_BLK, NCC * TOPK), lambda b: (b, 0)),
            pl.BlockSpec((K2_BLK, NCC * TOPK), lambda b: (b, 0)),
        ],
        out_shape=[
            jax.ShapeDtypeStruct((N_ENT, NCC * TOPK), jnp.int32),
            jax.ShapeDtypeStruct((N_ENT, NCC * TOPK), jnp.int32),
            jax.ShapeDtypeStruct((N_ENT, NCC * TOPK), f32),
        ],
    )(q, nsnd, src2d, rel2d)

    h_flat = h_all.reshape(NCC * N_ENT, H)
    gh = _sc_gather1(h_flat, ss.reshape(-1))

    grid3 = N_ENT // K3_BLK
    fused, c1s, csqs, ccrs = pl.pallas_call(
        _k3_body,
        grid=(grid3,),
        in_specs=[
            pl.BlockSpec((K3_BLK * NCC * TOPK, H), lambda b: (b, 0)),
            pl.BlockSpec((K3_BLK, NCC * TOPK), lambda b: (b, 0)),
            pl.BlockSpec((K3_BLK, NCC * TOPK), lambda b: (b, 0)),
            pl.BlockSpec((K3_BLK, H), lambda b: (b, 0)),
            pl.BlockSpec((MAX_N, NRE, H), lambda b: (0, 0, 0)),
            pl.BlockSpec((MAX_N, H, H), lambda b: (0, 0, 0)),
            pl.BlockSpec((1, 128), lambda b: (0, 0)),
        ],
        out_specs=[
            pl.BlockSpec((K3_BLK, H), lambda b: (b, 0)),
            pl.BlockSpec((8, H), lambda b: (0, 0)),
            pl.BlockSpec((8, H), lambda b: (0, 0)),
            pl.BlockSpec((8, H), lambda b: (0, 0)),
        ],
        out_shape=[
            jax.ShapeDtypeStruct((N_ENT, H), f32),
            jax.ShapeDtypeStruct((8, H), f32),
            jax.ShapeDtypeStruct((8, H), f32),
            jax.ShapeDtypeStruct((8, H), f32),
        ],
    )(gh, wsel, sr, ent_emb, r_tabs, neigh_w.astype(f32), eg_v)

    corr = jnp.float32(0.0)
    for i in range(MAX_N):
        mu1 = c1s[2 * i] / N_ENT
        mu2 = c1s[2 * i + 1] / N_ENT
        m12 = ccrs[i].sum() / (N_ENT * H) - (mu1 * mu2).mean()
        v1 = csqs[2 * i].sum() / (N_ENT * H) - (mu1 ** 2).mean()
        v2 = csqs[2 * i + 1].sum() / (N_ENT * H) - (mu2 ** 2).mean()
        corr_i = jnp.abs(m12) / (jnp.sqrt(v1) * jnp.sqrt(v2) + 1e-8)
        corr = corr + m[i] * corr_i
    corr = corr / m.sum()

    head, relg = _sc_gather2(fused, h_id, pred_rel, r_id, window=32)

    conv_mat = _build_conv_mat(conv_w.astype(f32))
    bconv = jnp.repeat(conv_b.astype(f32), CH * CW).reshape(1, FC_IN)
    fused_t = fused.T
    grid4 = BS // K4_BLK
    score = pl.pallas_call(
        _k4_body,
        grid=(grid4,),
        in_specs=[
            pl.BlockSpec((K4_BLK, H), lambda b: (b, 0)),
            pl.BlockSpec((K4_BLK, H), lambda b: (b, 0)),
            pl.BlockSpec((2 * KH * KW, FC_IN), lambda b: (0, 0)),
            pl.BlockSpec((1, FC_IN), lambda b: (0, 0)),
            pl.BlockSpec((FC_IN, H), lambda b: (0, 0)),
            pl.BlockSpec((1, H), lambda b: (0, 0)),
            pl.BlockSpec((H, N_ENT), lambda b: (0, 0)),
            pl.BlockSpec((1, N_ENT), lambda b: (0, 0)),
        ],
        out_specs=pl.BlockSpec((K4_BLK, N_ENT), lambda b: (b, 0)),
        out_shape=jax.ShapeDtypeStruct((BS, N_ENT), f32),
    )(head, relg, conv_mat, bconv, fc_w.astype(f32),
      fc_b.reshape(1, H).astype(f32), fused_t,
      score_b.reshape(1, N_ENT).astype(f32))

    return score, corr


# speedup vs baseline: 25.7467x; 1.0736x over previous
import dataclasses
import functools

import jax
import jax.numpy as jnp
from jax.experimental import pallas as pl
from jax.experimental.pallas import tpu as pltpu
from jax.experimental.pallas import tpu_sc as plsc

N_ENT = 10000
N_REL = 200
NRE = 2 * N_REL
H = 128
DEG = 32
TOPK = 8
MAX_N = 3
NCC = 2 * MAX_N
E = N_ENT * DEG
BS = 1024
OUT_CH = 32
KER = 7
KH = 8
KW = 16
CH = 2 * KH - KER + 1
CW = KW - KER + 1
FC_IN = OUT_CH * CH * CW

K1_BLK = 1000
K2_BLK = 400
K3_BLK = 400
K4_BLK = 256
GATHER_WIN = 128


def _leaky(x):
    return jnp.where(x > 0, x, 0.2 * x)


def _sc_compiler_params():
    cp = pltpu.CompilerParams()
    if "needs_layout_passes" in pltpu.CompilerParams.__dataclass_fields__:
        cp = dataclasses.replace(cp, needs_layout_passes=False)
    return cp


def _sc_gather1(table, idx, window=GATHER_WIN):
    n = idx.shape[0]
    vd = table.shape[1]
    info = plsc.get_sparse_core_info()
    nw = info.num_cores * info.num_subcores
    quantum = nw * window
    n_pad = ((n + quantum - 1) // quantum) * quantum
    if n_pad != n:
        idx = jnp.pad(idx, (0, n_pad - n))
    per_w = n_pad // nw
    nsteps = per_w // window
    mesh = plsc.VectorSubcoreMesh(core_axis_name="c", subcore_axis_name="s")

    @functools.partial(
        pl.kernel,
        out_type=jax.ShapeDtypeStruct((n_pad, vd), table.dtype),
        mesh=mesh,
        scratch_types=[
            pltpu.VMEM((window,), jnp.int32),
            pltpu.VMEM((window,), jnp.int32),
            pltpu.VMEM((window, vd), table.dtype),
            pltpu.VMEM((window, vd), table.dtype),
            pltpu.SemaphoreType.DMA,
            pltpu.SemaphoreType.DMA,
        ])
    def gk(t_hbm, i_hbm, o_hbm, i_v0, i_v1, r_v0, r_v1, sem0, sem1):
        wid = jax.lax.axis_index("s") * info.num_cores + jax.lax.axis_index("c")
        base = wid * per_w
        ivs = (i_v0, i_v1)
        rvs = (r_v0, r_v1)
        sems = (sem0, sem1)

        pltpu.sync_copy(i_hbm.at[pl.ds(base, window)], i_v0)
        pltpu.make_async_copy(t_hbm.at[i_v0], r_v0, sem0).start()

        @pl.loop(0, nsteps)
        def _(step):
            for par in range(2):
                @pl.when(jax.lax.rem(step, 2) == par)
                def _():
                    cb = base + step * window

                    @pl.when(step + 1 < nsteps)
                    def _():
                        nxt = 1 - par
                        pltpu.sync_copy(
                            i_hbm.at[pl.ds(cb + window, window)], ivs[nxt])
                        pltpu.make_async_copy(
                            t_hbm.at[ivs[nxt]], rvs[nxt], sems[nxt]).start()

                    pltpu.make_async_copy(
                        t_hbm.at[ivs[par]], rvs[par], sems[par]).wait()
                    pltpu.sync_copy(rvs[par], o_hbm.at[pl.ds(cb, window)])

    return gk(table, idx)


def _sc_scores(ns6, rs6, src, rel):
    n = src.shape[0]
    nwu = 25
    per_w = n // nwu
    chunk = 3200
    nsteps = per_w // chunk
    mesh = plsc.VectorSubcoreMesh(core_axis_name="c", subcore_axis_name="s")
    info = plsc.get_sparse_core_info()

    @functools.partial(
        pl.kernel,
        out_type=jax.ShapeDtypeStruct((nwu * NCC, per_w), jnp.float32),
        mesh=mesh,
        compiler_params=_sc_compiler_params(),
        scratch_types=(
            [pltpu.VMEM((N_ENT,), jnp.float32)] * NCC
            + [pltpu.VMEM((NRE,), jnp.float32)] * NCC
            + [pltpu.VMEM((chunk,), jnp.int32)] * 2
            + [pltpu.VMEM((chunk,), jnp.float32)] * NCC
        ))
    def sk(*args):
        (ns_hbm, rs_hbm, src_hbm, rel_hbm, q_hbm) = args[:5]
        sc = args[5:]
        ns_v = sc[:NCC]
        rs_v = sc[NCC:2 * NCC]
        src_v, rel_v = sc[2 * NCC:2 * NCC + 2]
        q_v = sc[2 * NCC + 2:]
        wid = jax.lax.axis_index("s") * info.num_cores + jax.lax.axis_index("c")

        @pl.when(wid < nwu)
        def _():
            for cc in range(NCC):
                pltpu.sync_copy(ns_hbm.at[cc], ns_v[cc])
                pltpu.sync_copy(rs_hbm.at[cc], rs_v[cc])

            @pl.loop(0, nsteps)
            def _(step):
                cbase = wid * per_w + step * chunk
                pltpu.sync_copy(src_hbm.at[pl.ds(cbase, chunk)], src_v)
                pltpu.sync_copy(rel_hbm.at[pl.ds(cbase, chunk)], rel_v)

                @pl.loop(0, chunk, step=16)
                def _(o):
                    sidx = src_v[pl.ds(o, 16)]
                    ridx = rel_v[pl.ds(o, 16)]
                    for cc in range(NCC):
                        qv = (plsc.load_gather(ns_v[cc], [sidx])
                              + plsc.load_gather(rs_v[cc], [ridx]))
                        q_v[cc][pl.ds(o, 16)] = qv

                for cc in range(NCC):
                    pltpu.sync_copy(
                        q_v[cc],
                        q_hbm.at[wid * NCC + cc, pl.ds(step * chunk, chunk)])

    q = sk(ns6, rs6, src, rel)
    return (q.reshape(nwu, NCC, per_w).transpose(1, 0, 2).reshape(NCC, n))


def _sc_gather2(tab1, idx1, tab2, idx2, window=GATHER_WIN):
    n = idx1.shape[0]
    vd1, vd2 = tab1.shape[1], tab2.shape[1]
    info = plsc.get_sparse_core_info()
    nw = info.num_cores * info.num_subcores
    per_w = n // nw
    assert n % (nw * window) == 0, (n, window)
    nsteps = per_w // window
    mesh = plsc.VectorSubcoreMesh(core_axis_name="c", subcore_axis_name="s")

    @functools.partial(
        pl.kernel,
        out_type=(jax.ShapeDtypeStruct((n, vd1), tab1.dtype),
                  jax.ShapeDtypeStruct((n, vd2), tab2.dtype)),
        mesh=mesh,
        scratch_types=[
            pltpu.VMEM((window,), jnp.int32),
            pltpu.VMEM((window, vd1), tab1.dtype),
            pltpu.VMEM((window,), jnp.int32),
            pltpu.VMEM((window, vd2), tab2.dtype),
        ])
    def gk(t1_hbm, i1_hbm, t2_hbm, i2_hbm, o1_hbm, o2_hbm,
           i1_v, r1_v, i2_v, r2_v):
        wid = jax.lax.axis_index("s") * info.num_cores + jax.lax.axis_index("c")
        base = wid * per_w

        @pl.loop(0, nsteps)
        def _(step):
            cb = base + step * window
            pltpu.sync_copy(i1_hbm.at[pl.ds(cb, window)], i1_v)
            pltpu.sync_copy(t1_hbm.at[i1_v], r1_v)
            pltpu.sync_copy(r1_v, o1_hbm.at[pl.ds(cb, window)])
            pltpu.sync_copy(i2_hbm.at[pl.ds(cb, window)], i2_v)
            pltpu.sync_copy(t2_hbm.at[i2_v], r2_v)
            pltpu.sync_copy(r2_v, o2_hbm.at[pl.ds(cb, window)])

    return gk(tab1, idx1, tab2, idx2)


def _k1_body(ent_ref, sw_ref, sb_ref, lw_ref, lb_ref, w_ref, wr_ref,
             a12_ref, a3_ref, re_ref, eg_ref,
             h_ref, nsnd_ref, rs_ref, rtab_ref, prel_ref):
    ent = ent_ref[...]
    common = jnp.dot(ent, sw_ref[...], preferred_element_type=jnp.float32) + sb_ref[...]
    private = jnp.dot(ent, lw_ref[...], preferred_element_type=jnp.float32) + lb_ref[...]
    ns_cols = []
    nd_cols = []
    for i in range(MAX_N):
        a12 = a12_ref[i]
        for ch, x in ((0, common), (1, private)):
            cc = 2 * i + ch
            h = jnp.dot(x, w_ref[i], preferred_element_type=jnp.float32)
            h_ref[cc, :, :] = h
            nsnd = jnp.dot(h, a12, preferred_element_type=jnp.float32)
            ns_cols.append(nsnd[:, 0:1])
            nd_cols.append(nsnd[:, 1:2])
    pad = jnp.zeros((ent.shape[0], 16 - 2 * NCC), jnp.float32)
    nsnd_ref[...] = jnp.concatenate(ns_cols + nd_cols + [pad], axis=1)

    @pl.when(pl.program_id(0) == 0)
    def _():
        rs_cols = []
        for i in range(MAX_N):
            rt = jnp.dot(re_ref[i], wr_ref[i], preferred_element_type=jnp.float32)
            rtab_ref[i, :, :] = rt
            rs = jnp.dot(rt, a3_ref[i], preferred_element_type=jnp.float32)
            rs_cols.append(rs)
            rs_cols.append(rs)
        rpad = jnp.zeros((NRE, 16 - NCC), jnp.float32)
        rs_ref[...] = jnp.concatenate(rs_cols + [rpad], axis=1)
        prel_ref[...] = (eg_ref[0, 0] * re_ref[0] + eg_ref[0, 1] * re_ref[1]
                         + eg_ref[0, 2] * re_ref[2])


def _k2_body(q_ref, nsnd_ref, src_ref, rel_ref,
             ss_ref, sr_ref, w_ref):
    blk = K2_BLK
    nsnd = nsnd_ref[...]
    src2d = src_ref[...]
    rel2d = rel_ref[...]
    ss_cols = []
    sr_cols = []
    w_cols = []
    lane = jax.lax.broadcasted_iota(jnp.int32, (blk, DEG), 1)
    for cc in range(NCC):
        s = _leaky(q_ref[cc] + nsnd[:, NCC + cc][:, None])
        rank = jnp.zeros((blk, DEG), jnp.int32)
        for k in range(DEG):
            sk = s[:, k:k + 1]
            beats = (sk > s) | ((sk == s) & (k < lane))
            rank = rank + beats.astype(jnp.int32)
        sel = rank < TOPK
        sm = jnp.where(sel, s, -1e30)
        mx = sm.max(axis=1, keepdims=True)
        p = jnp.where(sel, jnp.exp(sm - mx), 0.0)
        attn = p / p.sum(axis=1, keepdims=True)
        w8 = []
        ss8 = []
        sr8 = []
        for k in range(TOPK):
            oh = rank == k
            w8.append(jnp.where(oh, attn, 0.0).sum(axis=1, keepdims=True))
            ss8.append(jnp.where(oh, src2d, 0).sum(axis=1, keepdims=True))
            sr8.append(jnp.where(oh, rel2d, 0).sum(axis=1, keepdims=True))
        ss_cols.append(jnp.concatenate(ss8, axis=1) + cc * N_ENT)
        sr_cols.append(jnp.concatenate(sr8, axis=1))
        w_cols.append(jnp.concatenate(w8, axis=1))
    ss_ref[...] = jnp.concatenate(ss_cols, axis=1)
    sr_ref[...] = jnp.concatenate(sr_cols, axis=1)
    w_ref[...] = jnp.concatenate(w_cols, axis=1)


def _k3_body(gh_ref, w_ref, sr_ref, ent_ref, rtab_ref, nw_ref, eg_ref,
             fused_ref, c1_ref, csq_ref, ccr_ref):
    blk = K3_BLK
    G = gh_ref[...].reshape(blk, NCC * TOPK, H)
    w = w_ref[...]
    sr = sr_ref[...]

    @pl.when(pl.program_id(0) == 0)
    def _():
        c1_ref[...] = jnp.zeros_like(c1_ref)
        csq_ref[...] = jnp.zeros_like(csq_ref)
        ccr_ref[...] = jnp.zeros_like(ccr_ref)

    fused = ent_ref[...]
    iota_r = jax.lax.broadcasted_iota(jnp.int32, (blk, NRE), 1)
    for i in range(MAX_N):
        couts = []
        for ch in range(2):
            cc = 2 * i + ch
            wcc = w[:, cc * TOPK:(cc + 1) * TOPK]
            neigh = (G[:, cc * TOPK:(cc + 1) * TOPK, :] * wcc[:, :, None]).sum(axis=1)
            A = jnp.zeros((blk, NRE), jnp.float32)
            for k in range(TOPK):
                col = cc * TOPK + k
                A = A + jnp.where(sr[:, col][:, None] == iota_r,
                                  w[:, col][:, None], 0.0)
            neigh = neigh + jnp.dot(A, rtab_ref[i], preferred_element_type=jnp.float32)
            cout = jnp.tanh(jnp.dot(neigh, nw_ref[i], preferred_element_type=jnp.float32))
            couts.append(cout)
            fused = fused + eg_ref[0, i] * cout
            c1_ref[cc:cc + 1, :] += cout.sum(axis=0, keepdims=True)
            csq_ref[cc:cc + 1, :] += (cout * cout).sum(axis=0, keepdims=True)
        ccr_ref[i:i + 1, :] += (couts[0] * couts[1]).sum(axis=0, keepdims=True)
    fused_ref[...] = fused


def _k4_body(head_ref, rel_ref, m_ref, bconv_ref, fcw_ref, fcb_ref,
             fusedt_ref, sb_ref, out_ref):
    x = jnp.concatenate([head_ref[...], rel_ref[...]], axis=1)
    y1 = jnp.maximum(
        jnp.dot(x, m_ref[...], preferred_element_type=jnp.float32) + bconv_ref[...], 0.0)
    y2 = jnp.maximum(
        jnp.dot(y1, fcw_ref[...], preferred_element_type=jnp.float32) + fcb_ref[...], 0.0)
    out_ref[...] = (jnp.dot(y2, fusedt_ref[...], preferred_element_type=jnp.float32)
                    + sb_ref[...])


def _build_conv_mat(conv_w):
    py = jnp.arange(2 * KH)[:, None]
    oy = jnp.arange(CH)[None, :]
    dy = py - oy
    px = jnp.arange(KW)[:, None]
    ox = jnp.arange(CW)[None, :]
    dx = px - ox
    ok = (dy >= 0) & (dy < KER)
    okx = (dx >= 0) & (dx < KER)
    wy = jnp.clip(dy, 0, KER - 1)
    wx = jnp.clip(dx, 0, KER - 1)
    M = conv_w[:, 0][:, wy][:, :, :, wx]
    M = M * (ok[None, :, :, None, None] & okx[None, None, None, :, :])
    M = M.transpose(1, 3, 0, 2, 4)
    return M.reshape(2 * KH * KW, OUT_CH * CH * CW)


def kernel(h_id, r_id, src, dst, rel_id, ent_emb, gate, rel_embs,
           S_w, S_b, L_w, L_b, W, W_r, a, neigh_w,
           conv_w, conv_b, fc_w, fc_b, score_b):
    f32 = jnp.float32
    ent_emb = ent_emb.astype(f32)
    src = src.astype(jnp.int32)
    rel_id = rel_id.astype(jnp.int32)
    h_id = h_id.astype(jnp.int32)
    r_id = r_id.astype(jnp.int32)

    gw = jax.nn.softmax(gate.astype(f32))
    mask = gw > 0.1
    eff = jnp.where(mask.any(), mask, jnp.arange(MAX_N) == jnp.argmax(gw))
    m = eff.astype(f32)
    eg = gw * m
    eg = eg / eg.sum()
    eg_v = jnp.zeros((1, 128), f32).at[0, :MAX_N].set(eg)

    a_m = a[:, :, 0].astype(f32)
    a12 = jnp.stack([a_m[:, :H], a_m[:, H:2 * H]], axis=2)
    a3 = a_m[:, 2 * H:][:, :, None]
    sb2 = S_b.reshape(1, H).astype(f32)
    lb2 = L_b.reshape(1, H).astype(f32)

    grid1 = N_ENT // K1_BLK
    h_all, nsnd, rs_tab, r_tabs, pred_rel = pl.pallas_call(
        _k1_body,
        grid=(grid1,),
        in_specs=[
            pl.BlockSpec((K1_BLK, H), lambda b: (b, 0)),
            pl.BlockSpec((H, H), lambda b: (0, 0)),
            pl.BlockSpec((1, H), lambda b: (0, 0)),
            pl.BlockSpec((H, H), lambda b: (0, 0)),
            pl.BlockSpec((1, H), lambda b: (0, 0)),
            pl.BlockSpec((MAX_N, H, H), lambda b: (0, 0, 0)),
            pl.BlockSpec((MAX_N, H, H), lambda b: (0, 0, 0)),
            pl.BlockSpec((MAX_N, H, 2), lambda b: (0, 0, 0)),
            pl.BlockSpec((MAX_N, H, 1), lambda b: (0, 0, 0)),
            pl.BlockSpec((MAX_N, NRE, H), lambda b: (0, 0, 0)),
            pl.BlockSpec((1, 128), lambda b: (0, 0)),
        ],
        out_specs=[
            pl.BlockSpec((NCC, K1_BLK, H), lambda b: (0, b, 0)),
            pl.BlockSpec((K1_BLK, 16), lambda b: (b, 0)),
            pl.BlockSpec((NRE, 16), lambda b: (0, 0)),
            pl.BlockSpec((MAX_N, NRE, H), lambda b: (0, 0, 0)),
            pl.BlockSpec((NRE, H), lambda b: (0, 0)),
        ],
        out_shape=[
            jax.ShapeDtypeStruct((NCC, N_ENT, H), f32),
            jax.ShapeDtypeStruct((N_ENT, 16), f32),
            jax.ShapeDtypeStruct((NRE, 16), f32),
            jax.ShapeDtypeStruct((MAX_N, NRE, H), f32),
            jax.ShapeDtypeStruct((NRE, H), f32),
        ],
    )(ent_emb, S_w.astype(f32), sb2, L_w.astype(f32), lb2,
      W.astype(f32), W_r.astype(f32), a12, a3, rel_embs.astype(f32), eg_v)

    ns6 = nsnd[:, :NCC].T
    rs6 = rs_tab[:, :NCC].T
    q = _sc_scores(ns6, rs6, src, rel_id).reshape(NCC, N_ENT, DEG)

    grid2 = N_ENT // K2_BLK
    src2d = src.reshape(N_ENT, DEG)
    rel2d = rel_id.reshape(N_ENT, DEG)
    ss, sr, wsel = pl.pallas_call(
        _k2_body,
        grid=(grid2,),
        in_specs=[
            pl.BlockSpec((NCC, K2_BLK, DEG), lambda b: (0, b, 0)),
            pl.BlockSpec((K2_BLK, 16), lambda b: (b, 0)),
            pl.BlockSpec((K2_BLK, DEG), lambda b: (b, 0)),
            pl.BlockSpec((K2_BLK, DEG), lambda b: (b, 0)),
        ],
        out_specs=[
            pl.BlockSpec((K2_BLK, NCC * TOPK), lambda b: (b, 0)),
            pl.BlockSpec((K2_BLK, NCC * TOPK), lambda b: (b, 0)),
            pl.BlockSpec((K2_BLK, NCC * TOPK), lambda b: (b, 0)),
        ],
        out_shape=[
            jax.ShapeDtypeStruct((N_ENT, NCC * TOPK), jnp.int32),
            jax.ShapeDtypeStruct((N_ENT, NCC * TOPK), jnp.int32),
            jax.ShapeDtypeStruct((N_ENT, NCC * TOPK), f32),
        ],
    )(q, nsnd, src2d, rel2d)

    h_flat = h_all.reshape(NCC * N_ENT, H)
    gh = _sc_gather1(h_flat, ss.reshape(-1))

    grid3 = N_ENT // K3_BLK
    fused, c1s, csqs, ccrs = pl.pallas_call(
        _k3_body,
        grid=(grid3,),
        in_specs=[
            pl.BlockSpec((K3_BLK * NCC * TOPK, H), lambda b: (b, 0)),
            pl.BlockSpec((K3_BLK, NCC * TOPK), lambda b: (b, 0)),
            pl.BlockSpec((K3_BLK, NCC * TOPK), lambda b: (b, 0)),
            pl.BlockSpec((K3_BLK, H), lambda b: (b, 0)),
            pl.BlockSpec((MAX_N, NRE, H), lambda b: (0, 0, 0)),
            pl.BlockSpec((MAX_N, H, H), lambda b: (0, 0, 0)),
            pl.BlockSpec((1, 128), lambda b: (0, 0)),
        ],
        out_specs=[
            pl.BlockSpec((K3_BLK, H), lambda b: (b, 0)),
            pl.BlockSpec((8, H), lambda b: (0, 0)),
            pl.BlockSpec((8, H), lambda b: (0, 0)),
            pl.BlockSpec((8, H), lambda b: (0, 0)),
        ],
        out_shape=[
            jax.ShapeDtypeStruct((N_ENT, H), f32),
            jax.ShapeDtypeStruct((8, H), f32),
            jax.ShapeDtypeStruct((8, H), f32),
            jax.ShapeDtypeStruct((8, H), f32),
        ],
    )(gh, wsel, sr, ent_emb, r_tabs, neigh_w.astype(f32), eg_v)

    corr = jnp.float32(0.0)
    for i in range(MAX_N):
        mu1 = c1s[2 * i] / N_ENT
        mu2 = c1s[2 * i + 1] / N_ENT
        m12 = ccrs[i].sum() / (N_ENT * H) - (mu1 * mu2).mean()
        v1 = csqs[2 * i].sum() / (N_ENT * H) - (mu1 ** 2).mean()
        v2 = csqs[2 * i + 1].sum() / (N_ENT * H) - (mu2 ** 2).mean()
        corr_i = jnp.abs(m12) / (jnp.sqrt(v1) * jnp.sqrt(v2) + 1e-8)
        corr = corr + m[i] * corr_i
    corr = corr / m.sum()

    head, relg = _sc_gather2(fused, h_id, pred_rel, r_id, window=32)

    conv_mat = _build_conv_mat(conv_w.astype(f32))
    bconv = jnp.repeat(conv_b.astype(f32), CH * CW).reshape(1, FC_IN)
    fused_t = fused.T
    grid4 = BS // K4_BLK
    score = pl.pallas_call(
        _k4_body,
        grid=(grid4,),
        in_specs=[
            pl.BlockSpec((K4_BLK, H), lambda b: (b, 0)),
            pl.BlockSpec((K4_BLK, H), lambda b: (b, 0)),
            pl.BlockSpec((2 * KH * KW, FC_IN), lambda b: (0, 0)),
            pl.BlockSpec((1, FC_IN), lambda b: (0, 0)),
            pl.BlockSpec((FC_IN, H), lambda b: (0, 0)),
            pl.BlockSpec((1, H), lambda b: (0, 0)),
            pl.BlockSpec((H, N_ENT), lambda b: (0, 0)),
            pl.BlockSpec((1, N_ENT), lambda b: (0, 0)),
        ],
        out_specs=pl.BlockSpec((K4_BLK, N_ENT), lambda b: (b, 0)),
        out_shape=jax.ShapeDtypeStruct((BS, N_ENT), f32),
    )(head, relg, conv_mat, bconv, fc_w.astype(f32),
      fc_b.reshape(1, H).astype(f32), fused_t,
      score_b.reshape(1, N_ENT).astype(f32))

    return score, corr


# speedup vs baseline: 28.8717x; 1.1214x over previous
import dataclasses
import functools

import jax
import jax.numpy as jnp
from jax.experimental import pallas as pl
from jax.experimental.pallas import tpu as pltpu
from jax.experimental.pallas import tpu_sc as plsc

N_ENT = 10000
N_REL = 200
NRE = 2 * N_REL
H = 128
DEG = 32
TOPK = 8
MAX_N = 3
NCC = 2 * MAX_N
E = N_ENT * DEG
BS = 1024
OUT_CH = 32
KER = 7
KH = 8
KW = 16
CH = 2 * KH - KER + 1
CW = KW - KER + 1
FC_IN = OUT_CH * CH * CW

K1_BLK = 1000
K2_BLK = 400
K3_BLK = 400
K4_BLK = 256
GATHER_WIN = 128


def _leaky(x):
    return jnp.where(x > 0, x, 0.2 * x)


def _sc_compiler_params():
    cp = pltpu.CompilerParams()
    if "needs_layout_passes" in pltpu.CompilerParams.__dataclass_fields__:
        cp = dataclasses.replace(cp, needs_layout_passes=False)
    return cp


def _sc_gather1(table, idx, window=GATHER_WIN):
    n = idx.shape[0]
    vd = table.shape[1]
    info = plsc.get_sparse_core_info()
    nw = info.num_cores * info.num_subcores
    quantum = nw * window
    n_pad = ((n + quantum - 1) // quantum) * quantum
    if n_pad != n:
        idx = jnp.pad(idx, (0, n_pad - n))
    per_w = n_pad // nw
    nsteps = per_w // window
    mesh = plsc.VectorSubcoreMesh(core_axis_name="c", subcore_axis_name="s")

    @functools.partial(
        pl.kernel,
        out_type=jax.ShapeDtypeStruct((n_pad, vd), table.dtype),
        mesh=mesh,
        scratch_types=[
            pltpu.VMEM((window,), jnp.int32),
            pltpu.VMEM((window,), jnp.int32),
            pltpu.VMEM((window, vd), table.dtype),
            pltpu.VMEM((window, vd), table.dtype),
            pltpu.SemaphoreType.DMA,
            pltpu.SemaphoreType.DMA,
        ])
    def gk(t_hbm, i_hbm, o_hbm, i_v0, i_v1, r_v0, r_v1, sem0, sem1):
        wid = jax.lax.axis_index("s") * info.num_cores + jax.lax.axis_index("c")
        base = wid * per_w
        ivs = (i_v0, i_v1)
        rvs = (r_v0, r_v1)
        sems = (sem0, sem1)

        pltpu.sync_copy(i_hbm.at[pl.ds(base, window)], i_v0)
        pltpu.make_async_copy(t_hbm.at[i_v0], r_v0, sem0).start()

        @pl.loop(0, nsteps)
        def _(step):
            for par in range(2):
                @pl.when(jax.lax.rem(step, 2) == par)
                def _():
                    cb = base + step * window

                    @pl.when(step + 1 < nsteps)
                    def _():
                        nxt = 1 - par
                        pltpu.sync_copy(
                            i_hbm.at[pl.ds(cb + window, window)], ivs[nxt])
                        pltpu.make_async_copy(
                            t_hbm.at[ivs[nxt]], rvs[nxt], sems[nxt]).start()

                    pltpu.make_async_copy(
                        t_hbm.at[ivs[par]], rvs[par], sems[par]).wait()
                    pltpu.sync_copy(rvs[par], o_hbm.at[pl.ds(cb, window)])

    return gk(table, idx)


def _sc_scores(ns6, rs6, src, rel):
    n = src.shape[0]
    nwu = 25
    per_w = n // nwu
    chunk = 3200
    nsteps = per_w // chunk
    mesh = plsc.VectorSubcoreMesh(core_axis_name="c", subcore_axis_name="s")
    info = plsc.get_sparse_core_info()

    @functools.partial(
        pl.kernel,
        out_type=jax.ShapeDtypeStruct((nwu * NCC, per_w), jnp.float32),
        mesh=mesh,
        compiler_params=_sc_compiler_params(),
        scratch_types=(
            [pltpu.VMEM((N_ENT,), jnp.float32)] * NCC
            + [pltpu.VMEM((NRE,), jnp.float32)] * NCC
            + [pltpu.VMEM((chunk,), jnp.int32)] * 2
            + [pltpu.VMEM((chunk,), jnp.float32)] * NCC
        ))
    def sk(*args):
        (ns_hbm, rs_hbm, src_hbm, rel_hbm, q_hbm) = args[:5]
        sc = args[5:]
        ns_v = sc[:NCC]
        rs_v = sc[NCC:2 * NCC]
        src_v, rel_v = sc[2 * NCC:2 * NCC + 2]
        q_v = sc[2 * NCC + 2:]
        wid = jax.lax.axis_index("s") * info.num_cores + jax.lax.axis_index("c")

        @pl.when(wid < nwu)
        def _():
            for cc in range(NCC):
                pltpu.sync_copy(ns_hbm.at[cc], ns_v[cc])
                pltpu.sync_copy(rs_hbm.at[cc], rs_v[cc])

            @pl.loop(0, nsteps)
            def _(step):
                cbase = wid * per_w + step * chunk
                pltpu.sync_copy(src_hbm.at[pl.ds(cbase, chunk)], src_v)
                pltpu.sync_copy(rel_hbm.at[pl.ds(cbase, chunk)], rel_v)

                @pl.loop(0, chunk, step=16)
                def _(o):
                    sidx = src_v[pl.ds(o, 16)]
                    ridx = rel_v[pl.ds(o, 16)]
                    for cc in range(NCC):
                        qv = (plsc.load_gather(ns_v[cc], [sidx])
                              + plsc.load_gather(rs_v[cc], [ridx]))
                        q_v[cc][pl.ds(o, 16)] = qv

                for cc in range(NCC):
                    pltpu.sync_copy(
                        q_v[cc],
                        q_hbm.at[wid * NCC + cc, pl.ds(step * chunk, chunk)])

    q = sk(ns6, rs6, src, rel)
    return (q.reshape(nwu, NCC, per_w).transpose(1, 0, 2).reshape(NCC, n))


def _k1_body(ent_ref, sw_ref, sb_ref, lw_ref, lb_ref, w_ref, wr_ref,
             a12_ref, a3_ref, re_ref, eg_ref,
             h_ref, nsnd_ref, rs_ref, rtab_ref, prel_ref):
    ent = ent_ref[...]
    common = jnp.dot(ent, sw_ref[...], preferred_element_type=jnp.float32) + sb_ref[...]
    private = jnp.dot(ent, lw_ref[...], preferred_element_type=jnp.float32) + lb_ref[...]
    ns_cols = []
    nd_cols = []
    for i in range(MAX_N):
        a12 = a12_ref[i]
        for ch, x in ((0, common), (1, private)):
            cc = 2 * i + ch
            h = jnp.dot(x, w_ref[i], preferred_element_type=jnp.float32)
            h_ref[cc, :, :] = h
            nsnd = jnp.dot(h, a12, preferred_element_type=jnp.float32)
            ns_cols.append(nsnd[:, 0:1])
            nd_cols.append(nsnd[:, 1:2])
    pad = jnp.zeros((ent.shape[0], 16 - 2 * NCC), jnp.float32)
    nsnd_ref[...] = jnp.concatenate(ns_cols + nd_cols + [pad], axis=1)

    @pl.when(pl.program_id(0) == 0)
    def _():
        rs_cols = []
        for i in range(MAX_N):
            rt = jnp.dot(re_ref[i], wr_ref[i], preferred_element_type=jnp.float32)
            rtab_ref[i, :, :] = rt
            rs = jnp.dot(rt, a3_ref[i], preferred_element_type=jnp.float32)
            rs_cols.append(rs)
            rs_cols.append(rs)
        rpad = jnp.zeros((NRE, 16 - NCC), jnp.float32)
        rs_ref[...] = jnp.concatenate(rs_cols + [rpad], axis=1)
        prel_ref[...] = (eg_ref[0, 0] * re_ref[0] + eg_ref[0, 1] * re_ref[1]
                         + eg_ref[0, 2] * re_ref[2])


def _k2_body(q_ref, nsnd_ref, src_ref, rel_ref,
             ss_ref, sr_ref, w_ref):
    blk = K2_BLK
    nsnd = nsnd_ref[...]
    src2d = src_ref[...]
    rel2d = rel_ref[...]
    ss_cols = []
    sr_cols = []
    w_cols = []
    lane = jax.lax.broadcasted_iota(jnp.int32, (blk, DEG), 1)
    for cc in range(NCC):
        s = _leaky(q_ref[cc] + nsnd[:, NCC + cc][:, None])
        cur = s
        vals = []
        ss8 = []
        sr8 = []
        for k in range(TOPK):
            mx = cur.max(axis=1, keepdims=True)
            eq = cur == mx
            pos = jnp.where(eq, lane, DEG).min(axis=1, keepdims=True)
            first = lane == pos
            vals.append(mx)
            ss8.append(jnp.where(first, src2d, 0).sum(axis=1, keepdims=True))
            sr8.append(jnp.where(first, rel2d, 0).sum(axis=1, keepdims=True))
            cur = jnp.where(first, -jnp.inf, cur)
        v8 = jnp.concatenate(vals, axis=1)
        p = jnp.exp(v8 - v8[:, 0:1])
        w8 = p / p.sum(axis=1, keepdims=True)
        ss_cols.append(jnp.concatenate(ss8, axis=1) + cc * N_ENT)
        sr_cols.append(jnp.concatenate(sr8, axis=1))
        w_cols.append(w8)
    ss_ref[...] = jnp.concatenate(ss_cols, axis=1)
    sr_ref[...] = jnp.concatenate(sr_cols, axis=1)
    w_ref[...] = jnp.concatenate(w_cols, axis=1)


def _k3_body(gh_ref, w_ref, sr_ref, ent_ref, rtab_ref, nw_ref, eg_ref,
             fused_ref, c1_ref, csq_ref, ccr_ref):
    blk = K3_BLK
    G = gh_ref[...].reshape(blk, NCC * TOPK, H)
    w = w_ref[...]
    sr = sr_ref[...]

    @pl.when(pl.program_id(0) == 0)
    def _():
        c1_ref[...] = jnp.zeros_like(c1_ref)
        csq_ref[...] = jnp.zeros_like(csq_ref)
        ccr_ref[...] = jnp.zeros_like(ccr_ref)

    fused = ent_ref[...]
    iota_r = jax.lax.broadcasted_iota(jnp.int32, (blk, NRE), 1)
    for i in range(MAX_N):
        couts = []
        for ch in range(2):
            cc = 2 * i + ch
            wcc = w[:, cc * TOPK:(cc + 1) * TOPK]
            neigh = (G[:, cc * TOPK:(cc + 1) * TOPK, :] * wcc[:, :, None]).sum(axis=1)
            A = jnp.zeros((blk, NRE), jnp.float32)
            for k in range(TOPK):
                col = cc * TOPK + k
                A = A + jnp.where(sr[:, col][:, None] == iota_r,
                                  w[:, col][:, None], 0.0)
            neigh = neigh + jnp.dot(A, rtab_ref[i], preferred_element_type=jnp.float32)
            cout = jnp.tanh(jnp.dot(neigh, nw_ref[i], preferred_element_type=jnp.float32))
            couts.append(cout)
            fused = fused + eg_ref[0, i] * cout
            c1_ref[cc:cc + 1, :] += cout.sum(axis=0, keepdims=True)
            csq_ref[cc:cc + 1, :] += (cout * cout).sum(axis=0, keepdims=True)
        ccr_ref[i:i + 1, :] += (couts[0] * couts[1]).sum(axis=0, keepdims=True)
    fused_ref[...] = fused


def _k4_body(hid_ref, rid_ref, prel_ref, m_ref, bconv_ref, fcw_ref, fcb_ref,
             fused_ref, fusedt_ref, sb_ref, out_ref):
    blk = K4_BLK
    ent_iota = jax.lax.broadcasted_iota(jnp.int32, (blk, N_ENT), 1)
    hoh = (hid_ref[...] == ent_iota).astype(jnp.float32)
    head = jnp.dot(hoh, fused_ref[...], preferred_element_type=jnp.float32)
    rel_iota = jax.lax.broadcasted_iota(jnp.int32, (blk, NRE), 1)
    roh = (rid_ref[...] == rel_iota).astype(jnp.float32)
    rel = jnp.dot(roh, prel_ref[...], preferred_element_type=jnp.float32)
    x = jnp.concatenate([head, rel], axis=1)
    y1 = jnp.maximum(
        jnp.dot(x, m_ref[...], preferred_element_type=jnp.float32) + bconv_ref[...], 0.0)
    y2 = jnp.maximum(
        jnp.dot(y1, fcw_ref[...], preferred_element_type=jnp.float32) + fcb_ref[...], 0.0)
    out_ref[...] = (jnp.dot(y2, fusedt_ref[...], preferred_element_type=jnp.float32)
                    + sb_ref[...])


def _build_conv_mat(conv_w):
    py = jnp.arange(2 * KH)[:, None]
    oy = jnp.arange(CH)[None, :]
    dy = py - oy
    px = jnp.arange(KW)[:, None]
    ox = jnp.arange(CW)[None, :]
    dx = px - ox
    ok = (dy >= 0) & (dy < KER)
    okx = (dx >= 0) & (dx < KER)
    wy = jnp.clip(dy, 0, KER - 1)
    wx = jnp.clip(dx, 0, KER - 1)
    M = conv_w[:, 0][:, wy][:, :, :, wx]
    M = M * (ok[None, :, :, None, None] & okx[None, None, None, :, :])
    M = M.transpose(1, 3, 0, 2, 4)
    return M.reshape(2 * KH * KW, OUT_CH * CH * CW)


def kernel(h_id, r_id, src, dst, rel_id, ent_emb, gate, rel_embs,
           S_w, S_b, L_w, L_b, W, W_r, a, neigh_w,
           conv_w, conv_b, fc_w, fc_b, score_b):
    f32 = jnp.float32
    ent_emb = ent_emb.astype(f32)
    src = src.astype(jnp.int32)
    rel_id = rel_id.astype(jnp.int32)
    h_id = h_id.astype(jnp.int32)
    r_id = r_id.astype(jnp.int32)

    gw = jax.nn.softmax(gate.astype(f32))
    mask = gw > 0.1
    eff = jnp.where(mask.any(), mask, jnp.arange(MAX_N) == jnp.argmax(gw))
    m = eff.astype(f32)
    eg = gw * m
    eg = eg / eg.sum()
    eg_v = jnp.zeros((1, 128), f32).at[0, :MAX_N].set(eg)

    a_m = a[:, :, 0].astype(f32)
    a12 = jnp.stack([a_m[:, :H], a_m[:, H:2 * H]], axis=2)
    a3 = a_m[:, 2 * H:][:, :, None]
    sb2 = S_b.reshape(1, H).astype(f32)
    lb2 = L_b.reshape(1, H).astype(f32)

    grid1 = N_ENT // K1_BLK
    h_all, nsnd, rs_tab, r_tabs, pred_rel = pl.pallas_call(
        _k1_body,
        grid=(grid1,),
        in_specs=[
            pl.BlockSpec((K1_BLK, H), lambda b: (b, 0)),
            pl.BlockSpec((H, H), lambda b: (0, 0)),
            pl.BlockSpec((1, H), lambda b: (0, 0)),
            pl.BlockSpec((H, H), lambda b: (0, 0)),
            pl.BlockSpec((1, H), lambda b: (0, 0)),
            pl.BlockSpec((MAX_N, H, H), lambda b: (0, 0, 0)),
            pl.BlockSpec((MAX_N, H, H), lambda b: (0, 0, 0)),
            pl.BlockSpec((MAX_N, H, 2), lambda b: (0, 0, 0)),
            pl.BlockSpec((MAX_N, H, 1), lambda b: (0, 0, 0)),
            pl.BlockSpec((MAX_N, NRE, H), lambda b: (0, 0, 0)),
            pl.BlockSpec((1, 128), lambda b: (0, 0)),
        ],
        out_specs=[
            pl.BlockSpec((NCC, K1_BLK, H), lambda b: (0, b, 0)),
            pl.BlockSpec((K1_BLK, 16), lambda b: (b, 0)),
            pl.BlockSpec((NRE, 16), lambda b: (0, 0)),
            pl.BlockSpec((MAX_N, NRE, H), lambda b: (0, 0, 0)),
            pl.BlockSpec((NRE, H), lambda b: (0, 0)),
        ],
        out_shape=[
            jax.ShapeDtypeStruct((NCC, N_ENT, H), f32),
            jax.ShapeDtypeStruct((N_ENT, 16), f32),
            jax.ShapeDtypeStruct((NRE, 16), f32),
            jax.ShapeDtypeStruct((MAX_N, NRE, H), f32),
            jax.ShapeDtypeStruct((NRE, H), f32),
        ],
    )(ent_emb, S_w.astype(f32), sb2, L_w.astype(f32), lb2,
      W.astype(f32), W_r.astype(f32), a12, a3, rel_embs.astype(f32), eg_v)

    ns6 = nsnd[:, :NCC].T
    rs6 = rs_tab[:, :NCC].T
    q = _sc_scores(ns6, rs6, src, rel_id).reshape(NCC, N_ENT, DEG)

    grid2 = N_ENT // K2_BLK
    src2d = src.reshape(N_ENT, DEG)
    rel2d = rel_id.reshape(N_ENT, DEG)
    ss, sr, wsel = pl.pallas_call(
        _k2_body,
        grid=(grid2,),
        in_specs=[
            pl.BlockSpec((NCC, K2_BLK, DEG), lambda b: (0, b, 0)),
            pl.BlockSpec((K2_BLK, 16), lambda b: (b, 0)),
            pl.BlockSpec((K2_BLK, DEG), lambda b: (b, 0)),
            pl.BlockSpec((K2_BLK, DEG), lambda b: (b, 0)),
        ],
        out_specs=[
            pl.BlockSpec((K2_BLK, NCC * TOPK), lambda b: (b, 0)),
            pl.BlockSpec((K2_BLK, NCC * TOPK), lambda b: (b, 0)),
            pl.BlockSpec((K2_BLK, NCC * TOPK), lambda b: (b, 0)),
        ],
        out_shape=[
            jax.ShapeDtypeStruct((N_ENT, NCC * TOPK), jnp.int32),
            jax.ShapeDtypeStruct((N_ENT, NCC * TOPK), jnp.int32),
            jax.ShapeDtypeStruct((N_ENT, NCC * TOPK), f32),
        ],
    )(q, nsnd, src2d, rel2d)

    h_flat = h_all.reshape(NCC * N_ENT, H)
    gh = _sc_gather1(h_flat, ss.reshape(-1))

    grid3 = N_ENT // K3_BLK
    fused, c1s, csqs, ccrs = pl.pallas_call(
        _k3_body,
        grid=(grid3,),
        in_specs=[
            pl.BlockSpec((K3_BLK * NCC * TOPK, H), lambda b: (b, 0)),
            pl.BlockSpec((K3_BLK, NCC * TOPK), lambda b: (b, 0)),
            pl.BlockSpec((K3_BLK, NCC * TOPK), lambda b: (b, 0)),
            pl.BlockSpec((K3_BLK, H), lambda b: (b, 0)),
            pl.BlockSpec((MAX_N, NRE, H), lambda b: (0, 0, 0)),
            pl.BlockSpec((MAX_N, H, H), lambda b: (0, 0, 0)),
            pl.BlockSpec((1, 128), lambda b: (0, 0)),
        ],
        out_specs=[
            pl.BlockSpec((K3_BLK, H), lambda b: (b, 0)),
            pl.BlockSpec((8, H), lambda b: (0, 0)),
            pl.BlockSpec((8, H), lambda b: (0, 0)),
            pl.BlockSpec((8, H), lambda b: (0, 0)),
        ],
        out_shape=[
            jax.ShapeDtypeStruct((N_ENT, H), f32),
            jax.ShapeDtypeStruct((8, H), f32),
            jax.ShapeDtypeStruct((8, H), f32),
            jax.ShapeDtypeStruct((8, H), f32),
        ],
    )(gh, wsel, sr, ent_emb, r_tabs, neigh_w.astype(f32), eg_v)

    corr = jnp.float32(0.0)
    for i in range(MAX_N):
        mu1 = c1s[2 * i] / N_ENT
        mu2 = c1s[2 * i + 1] / N_ENT
        m12 = ccrs[i].sum() / (N_ENT * H) - (mu1 * mu2).mean()
        v1 = csqs[2 * i].sum() / (N_ENT * H) - (mu1 ** 2).mean()
        v2 = csqs[2 * i + 1].sum() / (N_ENT * H) - (mu2 ** 2).mean()
        corr_i = jnp.abs(m12) / (jnp.sqrt(v1) * jnp.sqrt(v2) + 1e-8)
        corr = corr + m[i] * corr_i
    corr = corr / m.sum()

    conv_mat = _build_conv_mat(conv_w.astype(f32))
    bconv = jnp.repeat(conv_b.astype(f32), CH * CW).reshape(1, FC_IN)
    fused_t = fused.T
    grid4 = BS // K4_BLK
    score = pl.pallas_call(
        _k4_body,
        grid=(grid4,),
        in_specs=[
            pl.BlockSpec((K4_BLK, 1), lambda b: (b, 0)),
            pl.BlockSpec((K4_BLK, 1), lambda b: (b, 0)),
            pl.BlockSpec((NRE, H), lambda b: (0, 0)),
            pl.BlockSpec((2 * KH * KW, FC_IN), lambda b: (0, 0)),
            pl.BlockSpec((1, FC_IN), lambda b: (0, 0)),
            pl.BlockSpec((FC_IN, H), lambda b: (0, 0)),
            pl.BlockSpec((1, H), lambda b: (0, 0)),
            pl.BlockSpec((N_ENT, H), lambda b: (0, 0)),
            pl.BlockSpec((H, N_ENT), lambda b: (0, 0)),
            pl.BlockSpec((1, N_ENT), lambda b: (0, 0)),
        ],
        out_specs=pl.BlockSpec((K4_BLK, N_ENT), lambda b: (b, 0)),
        out_shape=jax.ShapeDtypeStruct((BS, N_ENT), f32),
    )(h_id.reshape(BS, 1), r_id.reshape(BS, 1), pred_rel, conv_mat, bconv,
      fc_w.astype(f32), fc_b.reshape(1, H).astype(f32), fused, fused_t,
      score_b.reshape(1, N_ENT).astype(f32))

    return score, corr


# speedup vs baseline: 30.1284x; 1.0435x over previous
import dataclasses
import functools

import jax
import jax.numpy as jnp
from jax.experimental import pallas as pl
from jax.experimental.pallas import tpu as pltpu
from jax.experimental.pallas import tpu_sc as plsc

N_ENT = 10000
N_REL = 200
NRE = 2 * N_REL
H = 128
DEG = 32
TOPK = 8
MAX_N = 3
NCC = 2 * MAX_N
E = N_ENT * DEG
BS = 1024
OUT_CH = 32
KER = 7
KH = 8
KW = 16
CH = 2 * KH - KER + 1
CW = KW - KER + 1
FC_IN = OUT_CH * CH * CW

K1_BLK = 1000
K2_BLK = 400
K3_BLK = 400
K4_BLK = 256
GATHER_WIN = 128


def _leaky(x):
    return jnp.where(x > 0, x, 0.2 * x)


def _sc_compiler_params():
    cp = pltpu.CompilerParams()
    if "needs_layout_passes" in pltpu.CompilerParams.__dataclass_fields__:
        cp = dataclasses.replace(cp, needs_layout_passes=False)
    return cp


def _sc_gather1(table, idx, window=GATHER_WIN):
    n = idx.shape[0]
    vd = table.shape[1]
    info = plsc.get_sparse_core_info()
    nw = info.num_cores * info.num_subcores
    quantum = nw * window
    n_pad = ((n + quantum - 1) // quantum) * quantum
    if n_pad != n:
        idx = jnp.pad(idx, (0, n_pad - n))
    per_w = n_pad // nw
    nsteps = per_w // window
    mesh = plsc.VectorSubcoreMesh(core_axis_name="c", subcore_axis_name="s")

    @functools.partial(
        pl.kernel,
        out_type=jax.ShapeDtypeStruct((n_pad, vd), table.dtype),
        mesh=mesh,
        scratch_types=[
            pltpu.VMEM((window,), jnp.int32),
            pltpu.VMEM((window,), jnp.int32),
            pltpu.VMEM((window, vd), table.dtype),
            pltpu.VMEM((window, vd), table.dtype),
            pltpu.SemaphoreType.DMA,
            pltpu.SemaphoreType.DMA,
            pltpu.SemaphoreType.DMA,
            pltpu.SemaphoreType.DMA,
        ])
    def gk(t_hbm, i_hbm, o_hbm, i_v0, i_v1, r_v0, r_v1,
           gsem0, gsem1, ssem0, ssem1):
        assert nsteps % 2 == 0 and nsteps >= 4
        wid = jax.lax.axis_index("s") * info.num_cores + jax.lax.axis_index("c")
        base = wid * per_w
        ivs = (i_v0, i_v1)
        rvs = (r_v0, r_v1)
        gsems = (gsem0, gsem1)
        ssems = (ssem0, ssem1)

        pltpu.sync_copy(i_hbm.at[pl.ds(base, window)], i_v0)
        pltpu.make_async_copy(t_hbm.at[i_v0], r_v0, gsem0).start()

        @pl.loop(0, nsteps)
        def _(step):
            for par in range(2):
                @pl.when(jax.lax.rem(step, 2) == par)
                def _():
                    cb = base + step * window
                    nxt = 1 - par

                    @pl.when(step + 1 < nsteps)
                    def _():
                        @pl.when(step >= 1)
                        def _():
                            pltpu.make_async_copy(
                                rvs[nxt],
                                o_hbm.at[pl.ds(cb - window, window)],
                                ssems[nxt]).wait()
                        pltpu.sync_copy(
                            i_hbm.at[pl.ds(cb + window, window)], ivs[nxt])
                        pltpu.make_async_copy(
                            t_hbm.at[ivs[nxt]], rvs[nxt], gsems[nxt]).start()

                    pltpu.make_async_copy(
                        t_hbm.at[ivs[par]], rvs[par], gsems[par]).wait()
                    pltpu.make_async_copy(
                        rvs[par], o_hbm.at[pl.ds(cb, window)],
                        ssems[par]).start()

        for par in range(2):
            cb = base + (nsteps - 2 + par) * window
            pltpu.make_async_copy(
                rvs[par], o_hbm.at[pl.ds(cb, window)], ssems[par]).wait()

    return gk(table, idx)


def _sc_scores(ns6, rs6, src, rel):
    n = src.shape[0]
    nwu = 25
    per_w = n // nwu
    chunk = 3200
    nsteps = per_w // chunk
    mesh = plsc.VectorSubcoreMesh(core_axis_name="c", subcore_axis_name="s")
    info = plsc.get_sparse_core_info()

    @functools.partial(
        pl.kernel,
        out_type=jax.ShapeDtypeStruct((nwu * NCC, per_w), jnp.float32),
        mesh=mesh,
        compiler_params=_sc_compiler_params(),
        scratch_types=(
            [pltpu.VMEM((N_ENT,), jnp.float32)] * NCC
            + [pltpu.VMEM((NRE,), jnp.float32)] * NCC
            + [pltpu.VMEM((chunk,), jnp.int32)] * 2
            + [pltpu.VMEM((chunk,), jnp.float32)] * NCC
        ))
    def sk(*args):
        (ns_hbm, rs_hbm, src_hbm, rel_hbm, q_hbm) = args[:5]
        sc = args[5:]
        ns_v = sc[:NCC]
        rs_v = sc[NCC:2 * NCC]
        src_v, rel_v = sc[2 * NCC:2 * NCC + 2]
        q_v = sc[2 * NCC + 2:]
        wid = jax.lax.axis_index("s") * info.num_cores + jax.lax.axis_index("c")

        @pl.when(wid < nwu)
        def _():
            for cc in range(NCC):
                pltpu.sync_copy(ns_hbm.at[cc], ns_v[cc])
                pltpu.sync_copy(rs_hbm.at[cc], rs_v[cc])

            @pl.loop(0, nsteps)
            def _(step):
                cbase = wid * per_w + step * chunk
                pltpu.sync_copy(src_hbm.at[pl.ds(cbase, chunk)], src_v)
                pltpu.sync_copy(rel_hbm.at[pl.ds(cbase, chunk)], rel_v)

                @pl.loop(0, chunk, step=16)
                def _(o):
                    sidx = src_v[pl.ds(o, 16)]
                    ridx = rel_v[pl.ds(o, 16)]
                    for cc in range(NCC):
                        qv = (plsc.load_gather(ns_v[cc], [sidx])
                              + plsc.load_gather(rs_v[cc], [ridx]))
                        q_v[cc][pl.ds(o, 16)] = qv

                for cc in range(NCC):
                    pltpu.sync_copy(
                        q_v[cc],
                        q_hbm.at[wid * NCC + cc, pl.ds(step * chunk, chunk)])

    q = sk(ns6, rs6, src, rel)
    return q.reshape(nwu, NCC, per_w // DEG, DEG)


def _k1_body(ent_ref, sw_ref, sb_ref, lw_ref, lb_ref, w_ref, wr_ref,
             a12_ref, a3_ref, re_ref, eg_ref,
             h_ref, nsnd_ref, rs_ref, rtab_ref, prel_ref):
    ent = ent_ref[...]
    common = jnp.dot(ent, sw_ref[...], preferred_element_type=jnp.float32) + sb_ref[...]
    private = jnp.dot(ent, lw_ref[...], preferred_element_type=jnp.float32) + lb_ref[...]
    ns_cols = []
    nd_cols = []
    for i in range(MAX_N):
        a12 = a12_ref[i]
        for ch, x in ((0, common), (1, private)):
            cc = 2 * i + ch
            h = jnp.dot(x, w_ref[i], preferred_element_type=jnp.float32)
            h_ref[cc, :, :] = h
            nsnd = jnp.dot(h, a12, preferred_element_type=jnp.float32)
            ns_cols.append(nsnd[:, 0:1])
            nd_cols.append(nsnd[:, 1:2])
    pad = jnp.zeros((ent.shape[0], 16 - 2 * NCC), jnp.float32)
    nsnd_ref[...] = jnp.concatenate(ns_cols + nd_cols + [pad], axis=1)

    @pl.when(pl.program_id(0) == 0)
    def _():
        rs_cols = []
        for i in range(MAX_N):
            rt = jnp.dot(re_ref[i], wr_ref[i], preferred_element_type=jnp.float32)
            rtab_ref[i, :, :] = rt
            rs = jnp.dot(rt, a3_ref[i], preferred_element_type=jnp.float32)
            rs_cols.append(rs)
            rs_cols.append(rs)
        rpad = jnp.zeros((NRE, 16 - NCC), jnp.float32)
        rs_ref[...] = jnp.concatenate(rs_cols + [rpad], axis=1)
        prel_ref[...] = (eg_ref[0, 0] * re_ref[0] + eg_ref[0, 1] * re_ref[1]
                         + eg_ref[0, 2] * re_ref[2])


def _k2_body(q_ref, nsnd_ref, src_ref, rel_ref,
             ss_ref, sr_ref, w_ref):
    blk = K2_BLK
    nsnd = nsnd_ref[...]
    src2d = src_ref[...]
    rel2d = rel_ref[...]
    ss_cols = []
    sr_cols = []
    w_cols = []
    lane = jax.lax.broadcasted_iota(jnp.int32, (blk, DEG), 1)
    for cc in range(NCC):
        s = _leaky(q_ref[0, cc] + nsnd[:, NCC + cc][:, None])
        cur = s
        vals = []
        ss8 = []
        sr8 = []
        for k in range(TOPK):
            mx = cur.max(axis=1, keepdims=True)
            eq = cur == mx
            pos = jnp.where(eq, lane, DEG).min(axis=1, keepdims=True)
            first = lane == pos
            vals.append(mx)
            ss8.append(jnp.where(first, src2d, 0).sum(axis=1, keepdims=True))
            sr8.append(jnp.where(first, rel2d, 0).sum(axis=1, keepdims=True))
            cur = jnp.where(first, -jnp.inf, cur)
        v8 = jnp.concatenate(vals, axis=1)
        p = jnp.exp(v8 - v8[:, 0:1])
        w8 = p / p.sum(axis=1, keepdims=True)
        ss_cols.append(jnp.concatenate(ss8, axis=1) + cc * N_ENT)
        sr_cols.append(jnp.concatenate(sr8, axis=1))
        w_cols.append(w8)
    ss_ref[...] = jnp.concatenate(ss_cols, axis=1)
    sr_ref[...] = jnp.concatenate(sr_cols, axis=1)
    w_ref[...] = jnp.concatenate(w_cols, axis=1)


def _k3_body(gh_ref, w_ref, sr_ref, ent_ref, rtab_ref, nw_ref, eg_ref,
             fused_ref, c1_ref, csq_ref, ccr_ref):
    blk = K3_BLK
    G = gh_ref[...].reshape(blk, NCC * TOPK, H)
    w = w_ref[...]
    sr = sr_ref[...]

    @pl.when(pl.program_id(0) == 0)
    def _():
        c1_ref[...] = jnp.zeros_like(c1_ref)
        csq_ref[...] = jnp.zeros_like(csq_ref)
        ccr_ref[...] = jnp.zeros_like(ccr_ref)

    fused = ent_ref[...]
    iota_r = jax.lax.broadcasted_iota(jnp.int32, (blk, NRE), 1)
    for i in range(MAX_N):
        couts = []
        for ch in range(2):
            cc = 2 * i + ch
            wcc = w[:, cc * TOPK:(cc + 1) * TOPK]
            neigh = (G[:, cc * TOPK:(cc + 1) * TOPK, :] * wcc[:, :, None]).sum(axis=1)
            A = jnp.zeros((blk, NRE), jnp.float32)
            for k in range(TOPK):
                col = cc * TOPK + k
                A = A + jnp.where(sr[:, col][:, None] == iota_r,
                                  w[:, col][:, None], 0.0)
            neigh = neigh + jnp.dot(A, rtab_ref[i], preferred_element_type=jnp.float32)
            cout = jnp.tanh(jnp.dot(neigh, nw_ref[i], preferred_element_type=jnp.float32))
            couts.append(cout)
            fused = fused + eg_ref[0, i] * cout
            c1_ref[cc:cc + 1, :] += cout.sum(axis=0, keepdims=True)
            csq_ref[cc:cc + 1, :] += (cout * cout).sum(axis=0, keepdims=True)
        ccr_ref[i:i + 1, :] += (couts[0] * couts[1]).sum(axis=0, keepdims=True)
    fused_ref[...] = fused


def _k4_body(hid_ref, rid_ref, prel_ref, m_ref, bconv_ref, fcw_ref, fcb_ref,
             fused_ref, fusedt_ref, sb_ref, out_ref):
    blk = K4_BLK
    ent_iota = jax.lax.broadcasted_iota(jnp.int32, (blk, N_ENT), 1)
    hoh = (hid_ref[...] == ent_iota).astype(jnp.float32)
    head = jnp.dot(hoh, fused_ref[...], preferred_element_type=jnp.float32)
    rel_iota = jax.lax.broadcasted_iota(jnp.int32, (blk, NRE), 1)
    roh = (rid_ref[...] == rel_iota).astype(jnp.float32)
    rel = jnp.dot(roh, prel_ref[...], preferred_element_type=jnp.float32)
    x = jnp.concatenate([head, rel], axis=1)
    y1 = jnp.maximum(
        jnp.dot(x, m_ref[...], preferred_element_type=jnp.float32) + bconv_ref[...], 0.0)
    y2 = jnp.maximum(
        jnp.dot(y1, fcw_ref[...], preferred_element_type=jnp.float32) + fcb_ref[...], 0.0)
    out_ref[...] = (jnp.dot(y2, fusedt_ref[...], preferred_element_type=jnp.float32)
                    + sb_ref[...])


def _build_conv_mat(conv_w):
    py = jnp.arange(2 * KH)[:, None]
    oy = jnp.arange(CH)[None, :]
    dy = py - oy
    px = jnp.arange(KW)[:, None]
    ox = jnp.arange(CW)[None, :]
    dx = px - ox
    ok = (dy >= 0) & (dy < KER)
    okx = (dx >= 0) & (dx < KER)
    wy = jnp.clip(dy, 0, KER - 1)
    wx = jnp.clip(dx, 0, KER - 1)
    M = conv_w[:, 0][:, wy][:, :, :, wx]
    M = M * (ok[None, :, :, None, None] & okx[None, None, None, :, :])
    M = M.transpose(1, 3, 0, 2, 4)
    return M.reshape(2 * KH * KW, OUT_CH * CH * CW)


def kernel(h_id, r_id, src, dst, rel_id, ent_emb, gate, rel_embs,
           S_w, S_b, L_w, L_b, W, W_r, a, neigh_w,
           conv_w, conv_b, fc_w, fc_b, score_b):
    f32 = jnp.float32
    ent_emb = ent_emb.astype(f32)
    src = src.astype(jnp.int32)
    rel_id = rel_id.astype(jnp.int32)
    h_id = h_id.astype(jnp.int32)
    r_id = r_id.astype(jnp.int32)

    gw = jax.nn.softmax(gate.astype(f32))
    mask = gw > 0.1
    eff = jnp.where(mask.any(), mask, jnp.arange(MAX_N) == jnp.argmax(gw))
    m = eff.astype(f32)
    eg = gw * m
    eg = eg / eg.sum()
    eg_v = jnp.zeros((1, 128), f32).at[0, :MAX_N].set(eg)

    a_m = a[:, :, 0].astype(f32)
    a12 = jnp.stack([a_m[:, :H], a_m[:, H:2 * H]], axis=2)
    a3 = a_m[:, 2 * H:][:, :, None]
    sb2 = S_b.reshape(1, H).astype(f32)
    lb2 = L_b.reshape(1, H).astype(f32)

    grid1 = N_ENT // K1_BLK
    h_all, nsnd, rs_tab, r_tabs, pred_rel = pl.pallas_call(
        _k1_body,
        grid=(grid1,),
        in_specs=[
            pl.BlockSpec((K1_BLK, H), lambda b: (b, 0)),
            pl.BlockSpec((H, H), lambda b: (0, 0)),
            pl.BlockSpec((1, H), lambda b: (0, 0)),
            pl.BlockSpec((H, H), lambda b: (0, 0)),
            pl.BlockSpec((1, H), lambda b: (0, 0)),
            pl.BlockSpec((MAX_N, H, H), lambda b: (0, 0, 0)),
            pl.BlockSpec((MAX_N, H, H), lambda b: (0, 0, 0)),
            pl.BlockSpec((MAX_N, H, 2), lambda b: (0, 0, 0)),
            pl.BlockSpec((MAX_N, H, 1), lambda b: (0, 0, 0)),
            pl.BlockSpec((MAX_N, NRE, H), lambda b: (0, 0, 0)),
            pl.BlockSpec((1, 128), lambda b: (0, 0)),
        ],
        out_specs=[
            pl.BlockSpec((NCC, K1_BLK, H), lambda b: (0, b, 0)),
            pl.BlockSpec((K1_BLK, 16), lambda b: (b, 0)),
            pl.BlockSpec((NRE, 16), lambda b: (0, 0)),
            pl.BlockSpec((MAX_N, NRE, H), lambda b: (0, 0, 0)),
            pl.BlockSpec((NRE, H), lambda b: (0, 0)),
        ],
        out_shape=[
            jax.ShapeDtypeStruct((NCC, N_ENT, H), f32),
            jax.ShapeDtypeStruct((N_ENT, 16), f32),
            jax.ShapeDtypeStruct((NRE, 16), f32),
            jax.ShapeDtypeStruct((MAX_N, NRE, H), f32),
            jax.ShapeDtypeStruct((NRE, H), f32),
        ],
    )(ent_emb, S_w.astype(f32), sb2, L_w.astype(f32), lb2,
      W.astype(f32), W_r.astype(f32), a12, a3, rel_embs.astype(f32), eg_v)

    ns6 = nsnd[:, :NCC].T
    rs6 = rs_tab[:, :NCC].T
    q = _sc_scores(ns6, rs6, src, rel_id)

    grid2 = N_ENT // K2_BLK
    src2d = src.reshape(N_ENT, DEG)
    rel2d = rel_id.reshape(N_ENT, DEG)
    ss, sr, wsel = pl.pallas_call(
        _k2_body,
        grid=(grid2,),
        in_specs=[
            pl.BlockSpec((1, NCC, K2_BLK, DEG), lambda b: (b, 0, 0, 0)),
            pl.BlockSpec((K2_BLK, 16), lambda b: (b, 0)),
            pl.BlockSpec((K2_BLK, DEG), lambda b: (b, 0)),
            pl.BlockSpec((K2_BLK, DEG), lambda b: (b, 0)),
        ],
        out_specs=[
            pl.BlockSpec((K2_BLK, NCC * TOPK), lambda b: (b, 0)),
            pl.BlockSpec((K2_BLK, NCC * TOPK), lambda b: (b, 0)),
            pl.BlockSpec((K2_BLK, NCC * TOPK), lambda b: (b, 0)),
        ],
        out_shape=[
            jax.ShapeDtypeStruct((N_ENT, NCC * TOPK), jnp.int32),
            jax.ShapeDtypeStruct((N_ENT, NCC * TOPK), jnp.int32),
            jax.ShapeDtypeStruct((N_ENT, NCC * TOPK), f32),
        ],
    )(q, nsnd, src2d, rel2d)

    h_flat = h_all.reshape(NCC * N_ENT, H)
    gh = _sc_gather1(h_flat, ss.reshape(-1))

    grid3 = N_ENT // K3_BLK
    fused, c1s, csqs, ccrs = pl.pallas_call(
        _k3_body,
        grid=(grid3,),
        in_specs=[
            pl.BlockSpec((K3_BLK * NCC * TOPK, H), lambda b: (b, 0)),
            pl.BlockSpec((K3_BLK, NCC * TOPK), lambda b: (b, 0)),
            pl.BlockSpec((K3_BLK, NCC * TOPK), lambda b: (b, 0)),
            pl.BlockSpec((K3_BLK, H), lambda b: (b, 0)),
            pl.BlockSpec((MAX_N, NRE, H), lambda b: (0, 0, 0)),
            pl.BlockSpec((MAX_N, H, H), lambda b: (0, 0, 0)),
            pl.BlockSpec((1, 128), lambda b: (0, 0)),
        ],
        out_specs=[
            pl.BlockSpec((K3_BLK, H), lambda b: (b, 0)),
            pl.BlockSpec((8, H), lambda b: (0, 0)),
            pl.BlockSpec((8, H), lambda b: (0, 0)),
            pl.BlockSpec((8, H), lambda b: (0, 0)),
        ],
        out_shape=[
            jax.ShapeDtypeStruct((N_ENT, H), f32),
            jax.ShapeDtypeStruct((8, H), f32),
            jax.ShapeDtypeStruct((8, H), f32),
            jax.ShapeDtypeStruct((8, H), f32),
        ],
    )(gh, wsel, sr, ent_emb, r_tabs, neigh_w.astype(f32), eg_v)

    corr = jnp.float32(0.0)
    for i in range(MAX_N):
        mu1 = c1s[2 * i] / N_ENT
        mu2 = c1s[2 * i + 1] / N_ENT
        m12 = ccrs[i].sum() / (N_ENT * H) - (mu1 * mu2).mean()
        v1 = csqs[2 * i].sum() / (N_ENT * H) - (mu1 ** 2).mean()
        v2 = csqs[2 * i + 1].sum() / (N_ENT * H) - (mu2 ** 2).mean()
        corr_i = jnp.abs(m12) / (jnp.sqrt(v1) * jnp.sqrt(v2) + 1e-8)
        corr = corr + m[i] * corr_i
    corr = corr / m.sum()

    conv_mat = _build_conv_mat(conv_w.astype(f32))
    bconv = jnp.repeat(conv_b.astype(f32), CH * CW).reshape(1, FC_IN)
    fused_t = fused.T
    grid4 = BS // K4_BLK
    score = pl.pallas_call(
        _k4_body,
        grid=(grid4,),
        in_specs=[
            pl.BlockSpec((K4_BLK, 1), lambda b: (b, 0)),
            pl.BlockSpec((K4_BLK, 1), lambda b: (b, 0)),
            pl.BlockSpec((NRE, H), lambda b: (0, 0)),
            pl.BlockSpec((2 * KH * KW, FC_IN), lambda b: (0, 0)),
            pl.BlockSpec((1, FC_IN), lambda b: (0, 0)),
            pl.BlockSpec((FC_IN, H), lambda b: (0, 0)),
            pl.BlockSpec((1, H), lambda b: (0, 0)),
            pl.BlockSpec((N_ENT, H), lambda b: (0, 0)),
            pl.BlockSpec((H, N_ENT), lambda b: (0, 0)),
            pl.BlockSpec((1, N_ENT), lambda b: (0, 0)),
        ],
        out_specs=pl.BlockSpec((K4_BLK, N_ENT), lambda b: (b, 0)),
        out_shape=jax.ShapeDtypeStruct((BS, N_ENT), f32),
    )(h_id.reshape(BS, 1), r_id.reshape(BS, 1), pred_rel, conv_mat, bconv,
      fc_w.astype(f32), fc_b.reshape(1, H).astype(f32), fused, fused_t,
      score_b.reshape(1, N_ENT).astype(f32))

    return score, corr
